# baseline (speedup 1.0000x reference)
import functools

import jax
import jax.numpy as jnp
from jax import lax
from jax.experimental import pallas as pl
from jax.experimental.pallas import tpu as pltpu
from jax.experimental.pallas import tpu_sc as plsc

_NUM_PROMPTS = 1000
_SEQ_LEN = 77
_TEXT_DIM = 4096
_BATCH = 256

_NW = 32
_PPW = _BATCH // _NW
_RPW = _PPW * _SEQ_LEN
_CHUNK = 8
_NCHUNK = _RPW // _CHUNK
_NVEC = (_RPW + 15) // 16


def _sc_gather(idx, table):
    mesh = plsc.VectorSubcoreMesh(core_axis_name="c", subcore_axis_name="s")

    @functools.partial(
        pl.kernel,
        out_type=jax.ShapeDtypeStruct((_BATCH * _SEQ_LEN, _TEXT_DIM),
                                      jnp.float32),
        mesh=mesh,
        scratch_types=[
            pltpu.VMEM((16,), jnp.int32),
            pltpu.VMEM((_NVEC * 16,), jnp.int32),
            pltpu.VMEM((2, _CHUNK, _TEXT_DIM), jnp.float32),
            pltpu.SemaphoreType.DMA((2,)),
        ],
    )
    def k(idx_hbm, table_hbm, out_hbm, pids, rows, buf, sem):
        w = lax.axis_index("s") * 2 + lax.axis_index("c")
        base_p = w * _PPW
        base_r = w * _RPW

        pltpu.sync_copy(idx_hbm.at[pl.ds(base_p, _PPW)], pids.at[pl.ds(0, _PPW)])
        pv16 = pids[...]

        for j in range(_NVEC):
            r = lax.iota(jnp.int32, 16) + (j * 16)
            p = lax.div(r, _SEQ_LEN)
            s = r - p * _SEQ_LEN
            p = jnp.minimum(p, _PPW - 1)
            pv = pv16.at[p].get(mode="promise_in_bounds")
            rows[pl.ds(j * 16, 16)] = pv * _SEQ_LEN + s

        def start(c):
            par = lax.rem(c, 2)
            off = pl.multiple_of(c * _CHUNK, _CHUNK)
            pltpu.async_copy(
                table_hbm.at[rows.at[pl.ds(off, _CHUNK)]],
                buf.at[par], sem.at[par])

        start(0)

        def body(c, carry):
            par = lax.rem(c, 2)

            @pl.when(c + 1 < _NCHUNK)
            def _():
                start(c + 1)

            pltpu.make_async_copy(
                table_hbm.at[rows.at[pl.ds(0, _CHUNK)]],
                buf.at[par], sem.at[par]).wait()
            off = pl.multiple_of(c * _CHUNK, _CHUNK)
            pltpu.sync_copy(buf.at[par],
                            out_hbm.at[pl.ds(base_r + off, _CHUNK)])
            return carry

        lax.fori_loop(0, _NCHUNK, body, 0)

    return k(idx, table)


def kernel(prompt_idx, embeddings):
    idx = prompt_idx.astype(jnp.int32)
    table = embeddings.reshape(_NUM_PROMPTS * _SEQ_LEN, _TEXT_DIM)
    out = _sc_gather(idx, table)
    return out.reshape(_BATCH, _SEQ_LEN, _TEXT_DIM)


# speedup vs baseline: 5.0625x; 5.0625x over previous
import functools

import jax
import jax.numpy as jnp
from jax import lax
from jax.experimental import pallas as pl
from jax.experimental.pallas import tpu as pltpu
from jax.experimental.pallas import tpu_sc as plsc

_NUM_PROMPTS = 1000
_SEQ_LEN = 77
_TEXT_DIM = 4096
_BATCH = 256

_NW = 32
_PPW = _BATCH // _NW
_KPP = 10
_NCH = _PPW * _KPP
_NB = 3


def _chunk(i):
    p, kk = divmod(i, _KPP)
    return p, kk * 8, 5 if kk == _KPP - 1 else 8


def _sc_gather(idx, table):
    mesh = plsc.VectorSubcoreMesh(core_axis_name="c", subcore_axis_name="s")

    @functools.partial(
        pl.kernel,
        out_type=jax.ShapeDtypeStruct((_BATCH, _SEQ_LEN, _TEXT_DIM),
                                      jnp.float32),
        mesh=mesh,
        scratch_types=[
            pltpu.VMEM((16,), jnp.int32),
            pltpu.VMEM((_NB, 8, _TEXT_DIM), jnp.float32),
            pltpu.SemaphoreType.DMA((_NB,)),
            pltpu.SemaphoreType.DMA((_NB,)),
        ],
    )
    def k(idx_hbm, table_hbm, out_hbm, pids, buf, gsem, wsem):
        w = lax.axis_index("s") * 2 + lax.axis_index("c")
        base_p = w * _PPW

        pltpu.sync_copy(idx_hbm.at[pl.ds(base_p, _PPW)],
                        pids.at[pl.ds(0, _PPW)])
        pv16 = pids[...]

        def start_g(i):
            p, r0, nr = _chunk(i)
            pltpu.async_copy(
                table_hbm.at[pv16[p], pl.ds(r0, nr)],
                buf.at[i % _NB, pl.ds(0, nr)], gsem.at[i % _NB])

        def wait_g(i):
            _, r0, nr = _chunk(i)
            pltpu.make_async_copy(
                table_hbm.at[0, pl.ds(r0, nr)],
                buf.at[i % _NB, pl.ds(0, nr)], gsem.at[i % _NB]).wait()

        def start_w(i):
            p, r0, nr = _chunk(i)
            pltpu.async_copy(
                buf.at[i % _NB, pl.ds(0, nr)],
                out_hbm.at[base_p + p, pl.ds(r0, nr)], wsem.at[i % _NB])

        def wait_w(i):
            p, r0, nr = _chunk(i)
            pltpu.make_async_copy(
                buf.at[i % _NB, pl.ds(0, nr)],
                out_hbm.at[0, pl.ds(r0, nr)], wsem.at[i % _NB]).wait()

        for i in range(_NB):
            start_g(i)
        for i in range(_NCH):
            wait_g(i)
            start_w(i)
            if i + _NB < _NCH:
                wait_w(i)
                start_g(i + _NB)
        for i in range(_NCH - _NB, _NCH):
            wait_w(i)

    return k(idx, table)


def kernel(prompt_idx, embeddings):
    idx = prompt_idx.astype(jnp.int32)
    return _sc_gather(idx, embeddings)
